# Initial kernel scaffold; baseline (speedup 1.0000x reference)
import jax
import jax.numpy as jnp
from jax.experimental import pallas as pl


def kernel(peptide_encoding, pocket_encoding, kernels):
    raise NotImplementedError("write your pallas kernel here")


# speedup vs baseline: 1.6120x; 1.6120x over previous
import jax
import jax.numpy as jnp
from jax import lax
from jax.experimental import pallas as pl
from jax.experimental.pallas import tpu as pltpu
from jax.experimental.pallas import tpu_sc as plsc

_FILTER = 9
_ALPHA = 20
_PEP_LEN = 15
_NUM_POCKET = 34
_OUT = _FILTER + _ALPHA - 1
_L = 16
_ROW = 2 * _L
_NW = 32
_PAD = _FILTER - 1


def _body(pep_hbm, poc_hbm, ker_hbm, out_hbm,
          pep_v, poc_v, ker_v, xpad_v, row_v, sem0, sem1, sem2):
  wid = lax.axis_index("s") * 2 + lax.axis_index("c")

  cp0 = pltpu.async_copy(pep_hbm, pep_v, sem0)
  cp1 = pltpu.async_copy(poc_hbm, poc_v, sem1)
  cp2 = pltpu.async_copy(ker_hbm, ker_v, sem2)
  cp0.wait()
  cp1.wait()
  cp2.wait()

  zeros = jnp.zeros((_L,), jnp.float32)
  lane = lax.iota(jnp.int32, _L)

  def do_pocket(pocket):
    j1 = lax.rem(pocket, _PEP_LEN)
    j2 = lax.rem(pocket + 7, _PEP_LEN)
    o1 = j1 * _ALPHA
    o2 = j2 * _ALPHA
    a = pep_v[pl.ds(o1, _L)] + pep_v[pl.ds(o2, _L)]
    b = pep_v[pl.ds(o1 + _L, _L)] + pep_v[pl.ds(o2 + _L, _L)]
    b = jnp.where(lane < _ALPHA - _L, b, 0.0)
    xpad_v[pl.ds(0, _L)] = zeros
    xpad_v[pl.ds(_PAD, _L)] = a
    xpad_v[pl.ds(_PAD + _L, _L)] = b
    pid = plsc.load_gather(poc_v, [jnp.broadcast_to(pocket, (_L,))])
    acc0 = zeros
    acc1 = zeros
    for t in range(_FILTER):
      tap = plsc.load_gather(ker_v, [pid, jnp.full((_L,), t, jnp.int32)])
      acc0 = acc0 + tap * xpad_v[pl.ds(_PAD - t, _L)]
      acc1 = acc1 + tap * xpad_v[pl.ds(_PAD + _L - t, _L)]
    row_v[pl.ds(0, _L)] = acc0
    row_v[pl.ds(_L, _L)] = acc1
    pltpu.sync_copy(row_v, out_hbm.at[pocket])

  do_pocket(wid)

  @pl.when(wid < _NUM_POCKET - _NW)
  def _():
    do_pocket(wid + _NW)


@jax.jit
def kernel(peptide_encoding, pocket_encoding, kernels):
  pep_flat = jnp.pad(peptide_encoding.reshape(-1), (0, _L + 4))
  poc = jnp.pad(pocket_encoding, (0, 48 - _NUM_POCKET))
  ker = jnp.pad(kernels, ((0, 0), (0, _L - _FILTER)))

  out = pl.kernel(
      _body,
      out_type=jax.ShapeDtypeStruct((_NUM_POCKET, _ROW), jnp.float32),
      mesh=plsc.VectorSubcoreMesh(core_axis_name="c", subcore_axis_name="s"),
      compiler_params=pltpu.CompilerParams(needs_layout_passes=False),
      scratch_types=[
          pltpu.VMEM((_PEP_LEN * _ALPHA + _L + 4,), jnp.float32),
          pltpu.VMEM((48,), jnp.int32),
          pltpu.VMEM((_ALPHA, _L), jnp.float32),
          pltpu.VMEM((_ALPHA + 2 * _PAD + 4,), jnp.float32),
          pltpu.VMEM((_ROW,), jnp.float32),
          pltpu.SemaphoreType.DMA,
          pltpu.SemaphoreType.DMA,
          pltpu.SemaphoreType.DMA,
      ],
  )(pep_flat, poc, ker)
  return out[:, :_OUT]


# speedup vs baseline: 1.6601x; 1.0299x over previous
import jax
import jax.numpy as jnp
from jax import lax
from jax.experimental import pallas as pl
from jax.experimental.pallas import tpu as pltpu
from jax.experimental.pallas import tpu_sc as plsc

_FILTER = 9
_ALPHA = 20
_PEP_LEN = 15
_NUM_POCKET = 34
_OUT = _FILTER + _ALPHA - 1
_L = 16
_PAD = _FILTER - 1
_NPAIR = _NUM_POCKET // 2

_POC_OFF = _PEP_LEN * _ALPHA + _L + 4
_KER_OFF = _POC_OFF + 48
_BUF = _KER_OFF + _ALPHA * _L


def _body(buf_hbm, out_hbm, buf_v, xpad_v, row2_v, sem):
  wid = lax.axis_index("s") * 2 + lax.axis_index("c")

  pltpu.async_copy(buf_hbm, buf_v, sem).wait()

  @pl.when(wid < _NPAIR)
  def _():
    zeros = jnp.zeros((_L,), jnp.float32)
    lane = lax.iota(jnp.int32, _L)

    for q in range(2):
      pocket = 2 * wid + q
      j1 = lax.rem(pocket, _PEP_LEN)
      j2 = lax.rem(pocket + 7, _PEP_LEN)
      o1 = j1 * _ALPHA
      o2 = j2 * _ALPHA
      a = buf_v[pl.ds(o1, _L)] + buf_v[pl.ds(o2, _L)]
      b = buf_v[pl.ds(o1 + _L, _L)] + buf_v[pl.ds(o2 + _L, _L)]
      b = jnp.where(lane < _ALPHA - _L, b, 0.0)
      xpad_v[pl.ds(0, _L)] = zeros
      xpad_v[pl.ds(_PAD, _L)] = a
      xpad_v[pl.ds(_PAD + _L, _L)] = b
      pid_f = plsc.load_gather(buf_v, [jnp.broadcast_to(_POC_OFF + pocket, (_L,))])
      pid = pid_f.astype(jnp.int32)
      acc0 = zeros
      acc1 = zeros
      for t in range(_FILTER):
        tap = plsc.load_gather(buf_v, [pid * _L + (_KER_OFF + t)])
        acc0 = acc0 + tap * xpad_v[pl.ds(_PAD - t, _L)]
        acc1 = acc1 + tap * xpad_v[pl.ds(_PAD + _L - t, _L)]
      base = q * _OUT
      row2_v[pl.ds(base, _L)] = acc0
      row2_v[pl.ds(base + _L, _L)] = acc1
    pltpu.sync_copy(row2_v.at[pl.ds(0, 2 * _OUT)],
                    out_hbm.at[pl.ds(2 * _OUT * wid, 2 * _OUT)])


@jax.jit
def kernel(peptide_encoding, pocket_encoding, kernels):
  poc_f = jnp.pad(pocket_encoding, (0, 48 - _NUM_POCKET)).astype(jnp.float32)
  buf = jnp.concatenate([
      peptide_encoding.reshape(-1),
      jnp.zeros((_L + 4,), jnp.float32),
      poc_f,
      jnp.pad(kernels, ((0, 0), (0, _L - _FILTER))).reshape(-1),
  ])

  out = pl.kernel(
      _body,
      out_type=jax.ShapeDtypeStruct((_NUM_POCKET * _OUT,), jnp.float32),
      mesh=plsc.VectorSubcoreMesh(core_axis_name="c", subcore_axis_name="s"),
      compiler_params=pltpu.CompilerParams(needs_layout_passes=False),
      scratch_types=[
          pltpu.VMEM((_BUF,), jnp.float32),
          pltpu.VMEM((_ALPHA + 2 * _PAD + 4,), jnp.float32),
          pltpu.VMEM((4 * _L,), jnp.float32),
          pltpu.SemaphoreType.DMA,
      ],
  )(buf)
  return out.reshape(_NUM_POCKET, _OUT)
